# baseline (speedup 1.0000x reference)
import functools

import jax
import jax.numpy as jnp
from jax import lax
from jax.experimental import pallas as pl
from jax.experimental.pallas import tpu as pltpu
from jax.experimental.pallas import tpu_sc as plsc

_B = 16384
_D = 32
_V = 1000000
_L = 16
_WCH = 8
_NRB = 8
_MAPN = 256
_STAGE_ROWS = _B + _L


def _gather_stage(idx, tt):
    info = plsc.get_sparse_core_info()
    NW = info.num_cores * info.num_subcores
    mesh = plsc.VectorSubcoreMesh(core_axis_name="c", subcore_axis_name="s")

    @functools.partial(
        pl.kernel,
        mesh=mesh,
        out_type=jax.ShapeDtypeStruct((_STAGE_ROWS, 128), jnp.float32),
        scratch_types=[
            pltpu.VMEM((_B,), jnp.int32),
            pltpu.VMEM((_B,), jnp.int32),
            pltpu.VMEM((_B,), jnp.int32),
            pltpu.VMEM((_MAPN,), jnp.int32),
            pltpu.VMEM((_MAPN,), jnp.int32),
            pltpu.VMEM((_MAPN + 32,), jnp.int32),
            pltpu.VMEM((_WCH, _D, 128), jnp.float32),
            pltpu.VMEM((_NRB, _L, 128), jnp.float32),
            pltpu.SemaphoreType.DMA,
            pltpu.SemaphoreType.DMA,
        ],
        compiler_params=pltpu.CompilerParams(needs_layout_passes=False),
    )
    def k1(tt_hbm, idx_hbm, stage_hbm, idx_all, hitv, hitj, mapv, cums,
           wlist, slabs, rowbufs, semf, sems):
        w = lax.axis_index("s") * info.num_cores + lax.axis_index("c")
        pltpu.sync_copy(idx_hbm, idx_all)
        iota = lax.iota(jnp.int32, _L)
        zeros = jnp.zeros((_L,), jnp.int32)
        for t in range(_MAPN // _L):
            mapv[pl.ds(t * _L, _L)] = zeros

        def pre(g, cnt_v):
            v = idx_all[pl.ds(g * _L, _L)]
            m = ((v >> 7) & 31) == w
            ones = jnp.where(m, 1, 0).astype(jnp.int32)
            pos = cnt_v + plsc.cumsum(ones) - 1
            plsc.store_scatter(hitv, [pos], v, mask=m)
            plsc.store_scatter(hitj, [pos], iota + g * _L, mask=m)
            plsc.store_scatter(mapv, [(v >> 12) & (_MAPN - 1)], ones, mask=m)
            return cnt_v + plsc.all_reduce_population_count(m)

        cnt_v = lax.fori_loop(0, _B // _L, pre, zeros)
        cnt = cnt_v[0]

        carry = zeros
        for t in range(_MAPN // _L):
            mv = mapv[pl.ds(t * _L, _L)]
            cs = plsc.cumsum(mv) + carry
            cums[pl.ds(t * _L, _L)] = cs
            plsc.store_scatter(wlist, [cs - 1], iota + t * _L, mask=(mv == 1))
            carry = jnp.full((_L,), cs[_L - 1], jnp.int32)
        nwin = carry[0]
        smax = (7812 - w) >> 5
        nvh = (cnt + _L - 1) >> 4
        nchunk = (nwin + _WCH - 1) // _WCH

        def chunk(q, ts0):
            wv = jnp.minimum(wlist[pl.ds(q * _WCH, _L)] & (_MAPN - 1), smax)
            for jj in range(_WCH):
                row = pl.multiple_of(wv[jj] * 4096 + w * 128, 128)
                pltpu.async_copy(tt_hbm.at[:, pl.ds(row, 128)],
                                 slabs.at[jj], semf)
            for jj in range(_WCH):
                pltpu.make_async_copy(tt_hbm.at[:, pl.ds(0, 128)],
                                      slabs.at[0], semf).wait()

            def scan(e, ts1):
                v = hitv[pl.ds(e * _L, _L)]
                hj = hitj[pl.ds(e * _L, _L)]
                valid = (iota + e * _L) < cnt
                slot = (v >> 12) & (_MAPN - 1)
                sp = plsc.load_gather(cums, [slot]) - 1
                m = valid & (sp >= q * _WCH) & (sp < q * _WCH + _WCH)
                npc = plsc.all_reduce_population_count(m)

                def do(ts2):
                    sj = jnp.clip(sp - q * _WCH, 0, _WCH - 1)
                    lane = v & 127
                    jtar = jnp.where(m, hj, _B + iota)

                    @pl.when(ts2 >= _NRB)
                    def _():
                        pltpu.make_async_copy(
                            rowbufs.at[0], stage_hbm.at[pl.ds(0, _L)], sems
                        ).wait()

                    rb = rowbufs.at[ts2 % _NRB]
                    for j in range(_L):
                        spl = jnp.full((_L,), sj[j], jnp.int32)
                        lnl = jnp.full((_L,), lane[j], jnp.int32)
                        jrow = jnp.full((_L,), j, jnp.int32)
                        for h in range(_D // _L):
                            dvec = iota + h * _L
                            vals = plsc.load_gather(slabs, [spl, dvec, lnl])
                            plsc.store_scatter(rb, [jrow, dvec], vals)
                    pltpu.async_copy(rb, stage_hbm.at[jtar], sems)
                    return ts2 + 1

                return lax.cond(npc[0] > 0, do, lambda t: t, ts1)

            return lax.fori_loop(0, nvh, scan, ts0)

        ts = lax.fori_loop(0, nchunk, chunk, jnp.int32(0))

        def drain(i, acc):
            @pl.when(i < jnp.minimum(ts, _NRB))
            def _do_wait():
                pltpu.make_async_copy(
                    rowbufs.at[0], stage_hbm.at[pl.ds(0, _L)], sems
                ).wait()
            return acc

        lax.fori_loop(0, _NRB, drain, 0)

    return k1(tt, idx)


def _compact(stage):
    info = plsc.get_sparse_core_info()
    NW = info.num_cores * info.num_subcores
    bpw = _B // NW
    mesh = plsc.VectorSubcoreMesh(core_axis_name="c", subcore_axis_name="s")

    @functools.partial(
        pl.kernel,
        mesh=mesh,
        out_type=jax.ShapeDtypeStruct((_D, _B), jnp.float32),
        scratch_types=[
            pltpu.VMEM((bpw, 128), jnp.float32),
            pltpu.VMEM((_D, bpw), jnp.float32),
            pltpu.SemaphoreType.DMA,
        ],
        compiler_params=pltpu.CompilerParams(needs_layout_passes=False),
    )
    def k2(stage_hbm, out_hbm, sv, col, sem):
        wid = lax.axis_index("s") * info.num_cores + lax.axis_index("c")
        base = wid * bpw
        pltpu.sync_copy(stage_hbm.at[pl.ds(base, bpw)], sv)
        iota = lax.iota(jnp.int32, _L)

        def grp(g, _):
            j16 = iota + g * _L
            for d in range(_D):
                dspl = jnp.full((_L,), d, jnp.int32)
                vals = plsc.load_gather(sv, [j16, dspl])
                plsc.store_scatter(col, [dspl, j16], vals)
            return _

        lax.fori_loop(0, bpw // _L, grp, 0)
        pltpu.sync_copy(col, out_hbm.at[:, pl.ds(base, bpw)])

    return k2(stage)


def kernel(indices, embeddings):
    stage = _gather_stage(indices.astype(jnp.int32), embeddings.T)
    out_t = _compact(stage)
    return out_t.T


# speedup vs baseline: 4.4114x; 4.4114x over previous
import functools

import jax
import jax.numpy as jnp
from jax import lax
from jax.experimental import pallas as pl
from jax.experimental.pallas import tpu as pltpu
from jax.experimental.pallas import tpu_sc as plsc


def _lookup(idx, tt):
    (B,) = idx.shape
    D, V = tt.shape
    info = plsc.get_sparse_core_info()
    NW = info.num_cores * info.num_subcores
    L = info.num_lanes
    bpw = B // NW
    WAVE = 8
    nwave = bpw // WAVE
    mesh = plsc.VectorSubcoreMesh(core_axis_name="c", subcore_axis_name="s")

    @functools.partial(
        pl.kernel,
        mesh=mesh,
        out_type=jax.ShapeDtypeStruct((D, B), jnp.float32),
        scratch_types=[
            pltpu.VMEM((bpw + L,), jnp.int32),
            pltpu.VMEM((2, WAVE, D, 128), jnp.float32),
            pltpu.VMEM((D, bpw), jnp.float32),
            pltpu.SemaphoreType.DMA,
            pltpu.SemaphoreType.DMA,
        ],
        compiler_params=pltpu.CompilerParams(needs_layout_passes=False),
    )
    def k(tt_hbm, idx_hbm, out_hbm, idx_v, slab_v, col_v, sem0, sem1):
        wid = lax.axis_index("s") * info.num_cores + lax.axis_index("c")
        base = wid * bpw
        pltpu.sync_copy(idx_hbm.at[pl.ds(base, bpw)], idx_v.at[pl.ds(0, bpw)])
        sems = (sem0, sem1)

        def fire(g, p):
            v = idx_v[pl.ds(g * WAVE, L)]
            w = (v >> 7) << 7
            for j in range(WAVE):
                row = pl.multiple_of(w[j], 128)
                pltpu.async_copy(
                    tt_hbm.at[:, pl.ds(row, 128)], slab_v.at[p].at[j],
                    sems[p]
                )

        def drain(p):
            for j in range(WAVE):
                pltpu.make_async_copy(
                    tt_hbm.at[:, pl.ds(0, 128)], slab_v.at[0].at[0], sems[p]
                ).wait()

        def extract(g, p):
            v = idx_v[pl.ds(g * WAVE, L)]
            l = v & 127
            jbase = g * WAVE
            for j in range(WAVE):
                lane = jnp.full((L,), l[j], jnp.int32)
                sj = jnp.full((L,), j, jnp.int32)
                jcol = jnp.full((L,), jbase + j, jnp.int32)
                for h in range(D // L):
                    dvec = lax.iota(jnp.int32, L) + h * L
                    vals = plsc.load_gather(slab_v.at[p], [sj, dvec, lane])
                    plsc.store_scatter(col_v, [dvec, jcol], vals)

        fire(jnp.int32(0), 0)

        def pair(q, carry):
            g0 = 2 * q
            g1 = 2 * q + 1
            drain(0)
            fire(g1, 1)
            extract(g0, 0)
            drain(1)

            @pl.when(g1 + 1 < nwave)
            def _fire_next():
                fire(g1 + 1, 0)

            extract(g1, 1)
            return carry

        lax.fori_loop(0, nwave // 2, pair, 0)
        pltpu.sync_copy(col_v, out_hbm.at[:, pl.ds(base, bpw)])

    return k(tt, idx)


def kernel(indices, embeddings):
    out_t = _lookup(indices.astype(jnp.int32), embeddings.T)
    return out_t.T


# speedup vs baseline: 4.5783x; 1.0378x over previous
import functools

import jax
import jax.numpy as jnp
from jax import lax
from jax.experimental import pallas as pl
from jax.experimental.pallas import tpu as pltpu
from jax.experimental.pallas import tpu_sc as plsc


def _lookup(idx, tt):
    (B,) = idx.shape
    D, V = tt.shape
    info = plsc.get_sparse_core_info()
    NW = info.num_cores * info.num_subcores
    L = info.num_lanes
    bpw = B // NW
    WAVE = 16
    nwave = bpw // WAVE
    mesh = plsc.VectorSubcoreMesh(core_axis_name="c", subcore_axis_name="s")

    @functools.partial(
        pl.kernel,
        mesh=mesh,
        out_type=jax.ShapeDtypeStruct((D, B), jnp.float32),
        scratch_types=[
            pltpu.VMEM((bpw,), jnp.int32),
            pltpu.VMEM((WAVE, D, 128), jnp.float32),
            pltpu.VMEM((D, bpw), jnp.float32),
            pltpu.SemaphoreType.DMA,
        ],
        compiler_params=pltpu.CompilerParams(needs_layout_passes=False),
    )
    def k(tt_hbm, idx_hbm, out_hbm, idx_v, slab_v, col_v, sem):
        wid = lax.axis_index("s") * info.num_cores + lax.axis_index("c")
        base = wid * bpw
        pltpu.sync_copy(idx_hbm.at[pl.ds(base, bpw)], idx_v)

        def wave(g, carry):
            v = idx_v[pl.ds(g * WAVE, L)]
            w = (v >> 7) << 7
            l = v & 127
            for j in range(WAVE):
                row = pl.multiple_of(w[j], 128)
                pltpu.async_copy(
                    tt_hbm.at[:, pl.ds(row, 128)], slab_v.at[j], sem
                )
            for j in range(WAVE):
                pltpu.make_async_copy(
                    tt_hbm.at[:, pl.ds(0, 128)], slab_v.at[0], sem
                ).wait()
            jbase = g * WAVE
            for j in range(WAVE):
                lane = jnp.full((L,), l[j], jnp.int32)
                sj = jnp.full((L,), j, jnp.int32)
                jcol = jnp.full((L,), jbase + j, jnp.int32)
                for h in range(D // L):
                    dvec = lax.iota(jnp.int32, L) + h * L
                    vals = plsc.load_gather(slab_v, [sj, dvec, lane])
                    plsc.store_scatter(col_v, [dvec, jcol], vals)
            return carry

        lax.fori_loop(0, nwave, wave, 0)
        pltpu.sync_copy(col_v, out_hbm.at[:, pl.ds(base, bpw)])

    return k(tt, idx)


def kernel(indices, embeddings):
    out_t = _lookup(indices.astype(jnp.int32), embeddings.T)
    return out_t.T
